# baseline (speedup 1.0000x reference)
import functools

import jax
import jax.numpy as jnp
from jax import lax
from jax.experimental import pallas as pl
from jax.experimental.pallas import tpu as pltpu
from jax.experimental.pallas import tpu_sc as plsc

F32 = jnp.float32
BF16 = jnp.bfloat16

N = 10000
E = 320000
B = 16
DN = 128
DE = 16
DG = 16
DM = 64
HW = 80

CHUNK = 128
NWORK = 32
NSLICE = 4
ES = E // NSLICE

BN = 2000
BE = 2000

@functools.lru_cache(maxsize=1)
def _sc_mesh():
    return plsc.VectorSubcoreMesh(core_axis_name="c", subcore_axis_name="s")


def _k1_body(x_ref, batch_ref, u_ref, We1_ref, be1_ref, Wn1a_ref, bn1a_ref,
             T_ref, btbl_ref):
    xb = x_ref[...]
    bb = batch_ref[0, 0, :]
    oh = (bb[:, None] == lax.broadcasted_iota(jnp.int32, (BN, B), 1)).astype(F32)
    uw = jnp.dot(u_ref[...], We1_ref[2 * DN + DE:2 * DN + DE + DG, :],
                 preferred_element_type=F32)
    Ta = (jnp.dot(xb, We1_ref[0:DN, :], preferred_element_type=F32)
          + jnp.dot(oh, uw, preferred_element_type=F32) + be1_ref[...])
    Tp = jnp.dot(xb, Wn1a_ref[0:DN, :], preferred_element_type=F32) + bn1a_ref[...]
    T_ref[...] = jnp.concatenate([Ta, Tp], axis=1)
    bt = jnp.dot(xb, We1_ref[DN:2 * DN, :], preferred_element_type=F32)
    btbl_ref[...] = jnp.concatenate([bt, jnp.zeros((BN, DM), F32)], axis=1)


def _k1(x, batch3, u, We1, be1r, Wn1a, bn1ar):
    full = lambda shape: pl.BlockSpec(shape, lambda i: tuple(0 for _ in shape))
    return pl.pallas_call(
        _k1_body,
        grid=(N // BN,),
        in_specs=[
            pl.BlockSpec((BN, DN), lambda i: (i, 0)),
            pl.BlockSpec((1, 1, BN), lambda i: (i, 0, 0)),
            full((B, DG)),
            full((2 * DN + DE + DG, DM)),
            full((1, DM)),
            full((DM + DN, DM)),
            full((1, DM)),
        ],
        out_specs=[
            pl.BlockSpec((BN, 2 * DM), lambda i: (i, 0)),
            pl.BlockSpec((BN, 2 * DM), lambda i: (i, 0)),
        ],
        out_shape=[
            jax.ShapeDtypeStruct((N, 2 * DM), F32),
            jax.ShapeDtypeStruct((N, 2 * DM), F32),
        ],
    )(x, batch3, u, We1, be1r, Wn1a, bn1ar)


@functools.lru_cache(maxsize=4)
def _build_k2(ne):
    nchunk = ne // CHUNK
    nj = (nchunk + NWORK - 1) // NWORK

    @functools.partial(
        pl.kernel,
        mesh=_sc_mesh(),
        out_type=jax.ShapeDtypeStruct((ne, 2 * DM), F32),
        scratch_types=[
            pltpu.VMEM((CHUNK,), jnp.int32),
            pltpu.VMEM((CHUNK,), jnp.int32),
            pltpu.VMEM((CHUNK, 2 * DM), F32),
            pltpu.VMEM((CHUNK, 2 * DM), F32),
            pltpu.SemaphoreType.DMA,
            pltpu.SemaphoreType.DMA,
        ],
    )
    def _k2_body(T_hbm, btbl_hbm, row_hbm, col_hbm, G1_hbm,
                 rowi, coli, Tg, bg, s1, s2):
        wid = lax.axis_index("s") * 2 + lax.axis_index("c")

        def body(j, carry):
            c = wid + NWORK * j

            @pl.when(c < nchunk)
            def _():
                base = c * CHUNK
                pltpu.sync_copy(row_hbm.at[pl.ds(base, CHUNK)], rowi)
                pltpu.sync_copy(col_hbm.at[pl.ds(base, CHUNK)], coli)
                cp1 = pltpu.async_copy(T_hbm.at[rowi], Tg, s1)
                cp2 = pltpu.async_copy(btbl_hbm.at[coli], bg, s2)
                cp1.wait()
                cp2.wait()

                def add_row(i, carry2):
                    for q in range(DM // 16):
                        sl = pl.ds(q * 16, 16)
                        Tg[i, sl] += bg[i, sl]
                    return carry2

                lax.fori_loop(0, CHUNK, add_row, 0)
                pltpu.sync_copy(Tg, G1_hbm.at[pl.ds(base, CHUNK)])

            return carry

        lax.fori_loop(0, nj, body, 0)

    return _k2_body


def _k2_gather(T, btbl, row, col):
    return _build_k2(row.shape[0])(T, btbl, row, col)


def _k3_body(G1_ref, ea_ref, We1_ref, We2_ref, be2_ref, Wn1a_ref, _eacc_ref,
             eout_ref, h3_ref):
    g1 = G1_ref[...]
    h1 = jax.nn.relu(g1[:, 0:DM]
                     + jnp.dot(ea_ref[...], We1_ref[2 * DN:2 * DN + DE, :],
                               preferred_element_type=F32))
    eo = jnp.dot(h1, We2_ref[...], preferred_element_type=F32) + be2_ref[...]
    eout_ref[...] = eo
    h2 = jax.nn.relu(g1[:, DM:2 * DM]
                     + jnp.dot(eo, Wn1a_ref[DN:DN + DM, :],
                               preferred_element_type=F32))
    pat = (lax.broadcasted_iota(jnp.int32, (BE, HW - DM), 1) == 0).astype(F32)
    h3_ref[...] = jnp.concatenate([h2, jnp.broadcast_to(pat, (BE, HW - DM))],
                                  axis=1)


def _k3(G1, edge_attr, We1, We2, be2r, Wn1a, e_acc, s):
    ne = G1.shape[0]
    nb = ne // BE
    off = s * nb
    full = lambda shape: pl.BlockSpec(shape, lambda i: tuple(0 for _ in shape))
    return pl.pallas_call(
        _k3_body,
        grid=(nb,),
        in_specs=[
            pl.BlockSpec((BE, 2 * DM), lambda i: (i, 0)),
            pl.BlockSpec((BE, DE), lambda i: (i, 0)),
            full((2 * DN + DE + DG, DM)),
            full((DM, DM)),
            full((1, DM)),
            full((DM + DN, DM)),
            pl.BlockSpec((8, DM), lambda i: (0, 0)),
        ],
        out_specs=[
            pl.BlockSpec((BE, DM), lambda i: (i + off, 0)),
            pl.BlockSpec((BE, HW), lambda i: (i, 0)),
        ],
        out_shape=[
            jax.ShapeDtypeStruct((E, DM), F32),
            jax.ShapeDtypeStruct((ne, HW), F32),
        ],
        input_output_aliases={6: 0},
    )(G1, edge_attr, We1, We2, be2r, Wn1a, e_acc)


_ROWS_PER_TILE = N // 16


@functools.lru_cache(maxsize=4)
def _build_k4(ne):
    nchunk = ne // CHUNK
    nj = (nchunk + NWORK - 1) // NWORK

    @functools.partial(
        pl.kernel,
        mesh=_sc_mesh(),
        out_type=jax.ShapeDtypeStruct((2 * N, HW), F32),
        scratch_types=[
            pltpu.VMEM((CHUNK,), jnp.int32),
            pltpu.VMEM((CHUNK, HW), F32),
            pltpu.VMEM_SHARED((N, HW), F32),
            pltpu.SemaphoreType.DMA,
        ],
        compiler_params=pltpu.CompilerParams(use_tc_tiling_on_sc=False),
    )
    def _k4_body(h3_hbm, col_hbm, zero_hbm, S2_hbm, coli, h3v, shared, s1):
        cid = lax.axis_index("c")
        sid = lax.axis_index("s")
        wid = sid * 2 + cid

        pltpu.sync_copy(zero_hbm.at[pl.ds(sid * _ROWS_PER_TILE, _ROWS_PER_TILE)],
                        shared.at[pl.ds(sid * _ROWS_PER_TILE, _ROWS_PER_TILE)])
        plsc.subcore_barrier()

        def body(j, carry):
            c = wid + NWORK * j

            @pl.when(c < nchunk)
            def _():
                base = c * CHUNK
                pltpu.sync_copy(col_hbm.at[pl.ds(base, CHUNK)], coli)
                pltpu.sync_copy(h3_hbm.at[pl.ds(base, CHUNK)], h3v)
                pltpu.sync_copy(h3v, shared.at[coli], add=True)

            return carry

        lax.fori_loop(0, nj, body, 0)
        plsc.subcore_barrier()
        pltpu.sync_copy(shared.at[pl.ds(sid * _ROWS_PER_TILE, _ROWS_PER_TILE)],
                        S2_hbm.at[pl.ds(cid * N + sid * _ROWS_PER_TILE,
                                        _ROWS_PER_TILE)])

    return _k4_body


def _k4_scatter(h3, col, zeros):
    return _build_k4(col.shape[0])(h3, col, zeros)


def _k5_body(x_ref, *rest):
    nS = 2 * NSLICE
    S_refs = rest[:nS]
    (batch_ref, u_ref, Wn1b_ref, bn1b_ref, Wn2a_ref, bn2a_ref, Wn2b_ref,
     bn2b_ref, Wg1_ref, bg1_ref, Wg2_ref, bg2_ref,
     xout_ref, uout_ref, acc_ref) = rest[nS:]
    i = pl.program_id(0)
    S = S_refs[0][...]
    for r in S_refs[1:]:
        S = S + r[...]
    cnt = S[:, DM:DM + 1]
    hmean = S[:, 0:DM] / jnp.maximum(cnt, 1.0)
    agg = jnp.dot(hmean, Wn1b_ref[...], preferred_element_type=F32) + bn1b_ref[...]
    agg = jnp.where(cnt > 0.0, agg, 0.0)

    bb = batch_ref[0, 0, :]
    oh = (bb[:, None] == lax.broadcasted_iota(jnp.int32, (BN, B), 1)).astype(F32)
    uw = jnp.dot(u_ref[...], Wn2a_ref[DN + DM:DN + DM + DG, :],
                 preferred_element_type=F32)
    h = jax.nn.relu(jnp.dot(x_ref[...], Wn2a_ref[0:DN, :],
                            preferred_element_type=F32)
                    + jnp.dot(agg, Wn2a_ref[DN:DN + DM, :],
                              preferred_element_type=F32)
                    + jnp.dot(oh, uw, preferred_element_type=F32)
                    + bn2a_ref[...])
    xo = jnp.dot(h, Wn2b_ref[...], preferred_element_type=F32) + bn2b_ref[...]
    xout_ref[...] = xo

    xext = jnp.concatenate([xo, jnp.ones((BN, 1), F32)], axis=1)
    ps = lax.dot_general(oh, xext, (((0,), (0,)), ((), ())),
                         preferred_element_type=F32)

    @pl.when(i == 0)
    def _():
        acc_ref[...] = jnp.zeros_like(acc_ref)

    acc_ref[...] += ps

    @pl.when(i == N // BN - 1)
    def _():
        acc = acc_ref[...]
        g = acc[:, 0:DM] / jnp.maximum(acc[:, DM:DM + 1], 1.0)
        hu = jax.nn.relu(jnp.dot(u_ref[...], Wg1_ref[0:DG, :],
                                 preferred_element_type=F32)
                         + jnp.dot(g, Wg1_ref[DG:DG + DM, :],
                                   preferred_element_type=F32)
                         + bg1_ref[...])
        uout_ref[...] = jnp.dot(hu, Wg2_ref[...],
                                preferred_element_type=F32) + bg2_ref[...]


def _k5(x, S_parts, batch3, u, Wn1b, bn1br, Wn2a, bn2ar, Wn2b, bn2br,
        Wg1, bg1r, Wg2, bg2r):
    full = lambda shape: pl.BlockSpec(shape, lambda i: tuple(0 for _ in shape))
    return pl.pallas_call(
        _k5_body,
        grid=(N // BN,),
        in_specs=[
            pl.BlockSpec((BN, DN), lambda i: (i, 0)),
            *[pl.BlockSpec((BN, HW), lambda i: (i, 0)) for _ in S_parts],
            pl.BlockSpec((1, 1, BN), lambda i: (i, 0, 0)),
            full((B, DG)),
            full((DM, DM)),
            full((1, DM)),
            full((DN + DM + DG, DM)),
            full((1, DM)),
            full((DM, DM)),
            full((1, DM)),
            full((DG + DM, DM)),
            full((1, DM)),
            full((DM, DM)),
            full((1, DM)),
        ],
        out_specs=[
            pl.BlockSpec((BN, DM), lambda i: (i, 0)),
            pl.BlockSpec((B, DM), lambda i: (0, 0)),
        ],
        out_shape=[
            jax.ShapeDtypeStruct((N, DM), F32),
            jax.ShapeDtypeStruct((B, DM), F32),
        ],
        scratch_shapes=[pltpu.VMEM((B, DM + 1), F32)],
    )(x, *S_parts, batch3, u, Wn1b, bn1br, Wn2a, bn2ar, Wn2b, bn2br,
      Wg1, bg1r, Wg2, bg2r)


def kernel(x, edge_index, edge_attr, u, batch,
           We1, be1, We2, be2,
           Wn1a, bn1a, Wn1b, bn1b,
           Wn2a, bn2a, Wn2b, bn2b,
           Wg1, bg1, Wg2, bg2):
    row = edge_index[0]
    col = edge_index[1]
    batch3 = batch.astype(jnp.int32).reshape(N // BN, 1, BN)
    r1 = lambda b: b.reshape(1, DM)

    T, btbl = _k1(x, batch3, u, We1, r1(be1), Wn1a, r1(bn1a))
    zeros = jnp.zeros((N, HW), F32)
    row32 = row.astype(jnp.int32)
    col32 = col.astype(jnp.int32)
    S_parts = []
    e_out = jnp.zeros((E, DM), F32)
    for s in range(NSLICE):
        sl = slice(s * ES, (s + 1) * ES)
        G1 = _k2_gather(T, btbl, row32[sl], col32[sl])
        e_out, h3_s = _k3(G1, edge_attr[sl], We1, We2, r1(be2), Wn1a, e_out, s)
        S2_s = _k4_scatter(h3_s, col32[sl], zeros)
        S_parts.extend([S2_s[:N], S2_s[N:]])
    x_out, u_out = _k5(x, S_parts, batch3, u,
                       Wn1b, r1(bn1b), Wn2a, r1(bn2a), Wn2b, r1(bn2b),
                       Wg1, r1(bg1), Wg2, r1(bg2))
    return (x_out, e_out, u_out)


# speedup vs baseline: 1.0266x; 1.0266x over previous
import functools

import jax
import jax.numpy as jnp
from jax import lax
from jax.experimental import pallas as pl
from jax.experimental.pallas import tpu as pltpu
from jax.experimental.pallas import tpu_sc as plsc

F32 = jnp.float32
BF16 = jnp.bfloat16

N = 10000
E = 320000
B = 16
DN = 128
DE = 16
DG = 16
DM = 64
HW = 80

CHUNK = 128
NWORK = 32
NSLICE = 4
ES = E // NSLICE

BN = 2000
BE = 2000

@functools.lru_cache(maxsize=1)
def _sc_mesh():
    return plsc.VectorSubcoreMesh(core_axis_name="c", subcore_axis_name="s")


def _k1_body(x_ref, batch_ref, u_ref, We1_ref, be1_ref, Wn1a_ref, bn1a_ref,
             T_ref, btbl_ref):
    xb = x_ref[...]
    bb = batch_ref[0, 0, :]
    oh = (bb[:, None] == lax.broadcasted_iota(jnp.int32, (BN, B), 1)).astype(F32)
    uw = jnp.dot(u_ref[...], We1_ref[2 * DN + DE:2 * DN + DE + DG, :],
                 preferred_element_type=F32)
    Ta = (jnp.dot(xb, We1_ref[0:DN, :], preferred_element_type=F32)
          + jnp.dot(oh, uw, preferred_element_type=F32) + be1_ref[...])
    Tp = jnp.dot(xb, Wn1a_ref[0:DN, :], preferred_element_type=F32) + bn1a_ref[...]
    T_ref[...] = jnp.concatenate([Ta, Tp], axis=1)
    bt = jnp.dot(xb, We1_ref[DN:2 * DN, :], preferred_element_type=F32)
    btbl_ref[...] = jnp.concatenate([bt, jnp.zeros((BN, DM), F32)], axis=1)


def _k1(x, batch3, u, We1, be1r, Wn1a, bn1ar):
    full = lambda shape: pl.BlockSpec(shape, lambda i: tuple(0 for _ in shape))
    return pl.pallas_call(
        _k1_body,
        grid=(N // BN,),
        in_specs=[
            pl.BlockSpec((BN, DN), lambda i: (i, 0)),
            pl.BlockSpec((1, 1, BN), lambda i: (i, 0, 0)),
            full((B, DG)),
            full((2 * DN + DE + DG, DM)),
            full((1, DM)),
            full((DM + DN, DM)),
            full((1, DM)),
        ],
        out_specs=[
            pl.BlockSpec((BN, 2 * DM), lambda i: (i, 0)),
            pl.BlockSpec((BN, 2 * DM), lambda i: (i, 0)),
        ],
        out_shape=[
            jax.ShapeDtypeStruct((N, 2 * DM), F32),
            jax.ShapeDtypeStruct((N, 2 * DM), F32),
        ],
    )(x, batch3, u, We1, be1r, Wn1a, bn1ar)


@functools.lru_cache(maxsize=4)
def _build_k2(ne):
    nchunk = ne // CHUNK
    nj = (nchunk + NWORK - 1) // NWORK

    @functools.partial(
        pl.kernel,
        mesh=_sc_mesh(),
        out_type=jax.ShapeDtypeStruct((ne, 2 * DM), F32),
        scratch_types=[
            pltpu.VMEM((2, CHUNK), jnp.int32),
            pltpu.VMEM((2, CHUNK), jnp.int32),
            pltpu.VMEM((2, CHUNK, 2 * DM), F32),
            pltpu.VMEM((2, CHUNK, 2 * DM), F32),
            pltpu.SemaphoreType.DMA,
            pltpu.SemaphoreType.DMA,
            pltpu.SemaphoreType.DMA,
            pltpu.SemaphoreType.DMA,
        ],
    )
    def _k2_body(T_hbm, btbl_hbm, row_hbm, col_hbm, G1_hbm,
                 rowi, coli, Tg, bg, sT0, sT1, sb0, sb1):
        wid = lax.axis_index("s") * 2 + lax.axis_index("c")
        sT = (sT0, sT1)
        sb = (sb0, sb1)

        def start(j, b):
            c = wid + NWORK * j

            @pl.when(c < nchunk)
            def _():
                base = c * CHUNK
                pltpu.sync_copy(row_hbm.at[pl.ds(base, CHUNK)], rowi.at[b])
                pltpu.sync_copy(col_hbm.at[pl.ds(base, CHUNK)], coli.at[b])
                pltpu.async_copy(T_hbm.at[rowi.at[b]], Tg.at[b], sT[b])
                pltpu.async_copy(btbl_hbm.at[coli.at[b]], bg.at[b], sb[b])

        def finish(j, b):
            c = wid + NWORK * j

            @pl.when(c < nchunk)
            def _():
                base = c * CHUNK
                pltpu.make_async_copy(T_hbm.at[rowi.at[b]], Tg.at[b],
                                      sT[b]).wait()
                pltpu.make_async_copy(btbl_hbm.at[coli.at[b]], bg.at[b],
                                      sb[b]).wait()

                def add_row(i, carry2):
                    for q in range(DM // 16):
                        sl = pl.ds(q * 16, 16)
                        Tg[b, i, sl] += bg[b, i, sl]
                    return carry2

                lax.fori_loop(0, CHUNK, add_row, 0)
                pltpu.sync_copy(Tg.at[b], G1_hbm.at[pl.ds(base, CHUNK)])

        start(0, 0)

        def body(jj, carry):
            j0 = 2 * jj
            start(j0 + 1, 1)
            finish(j0, 0)
            start(j0 + 2, 0)
            finish(j0 + 1, 1)
            return carry

        lax.fori_loop(0, (nj + 1) // 2, body, 0)

    return _k2_body


def _k2_gather(T, btbl, row, col):
    return _build_k2(row.shape[0])(T, btbl, row, col)


def _k3_body(G1_ref, ea_ref, We1_ref, We2_ref, be2_ref, Wn1a_ref,
             eout_ref, h3_ref):
    g1 = G1_ref[...]
    h1 = jax.nn.relu(g1[:, 0:DM]
                     + jnp.dot(ea_ref[...], We1_ref[2 * DN:2 * DN + DE, :],
                               preferred_element_type=F32))
    eo = jnp.dot(h1, We2_ref[...], preferred_element_type=F32) + be2_ref[...]
    eout_ref[...] = eo
    h2 = jax.nn.relu(g1[:, DM:2 * DM]
                     + jnp.dot(eo, Wn1a_ref[DN:DN + DM, :],
                               preferred_element_type=F32))
    pat = (lax.broadcasted_iota(jnp.int32, (BE, HW - DM), 1) == 0).astype(F32)
    h3_ref[...] = jnp.concatenate([h2, jnp.broadcast_to(pat, (BE, HW - DM))],
                                  axis=1)


def _k3(G1, edge_attr, We1, We2, be2r, Wn1a, s):
    ne = G1.shape[0]
    nb = ne // BE
    full = lambda shape: pl.BlockSpec(shape, lambda i: tuple(0 for _ in shape))
    return pl.pallas_call(
        _k3_body,
        grid=(nb,),
        in_specs=[
            pl.BlockSpec((BE, 2 * DM), lambda i: (i, 0)),
            pl.BlockSpec((BE, DE), lambda i: (i, 0)),
            full((2 * DN + DE + DG, DM)),
            full((DM, DM)),
            full((1, DM)),
            full((DM + DN, DM)),
        ],
        out_specs=[
            pl.BlockSpec((BE, DM), lambda i: (i, 0)),
            pl.BlockSpec((BE, HW), lambda i: (i, 0)),
        ],
        out_shape=[
            jax.ShapeDtypeStruct((ne, DM), F32),
            jax.ShapeDtypeStruct((ne, HW), F32),
        ],
    )(G1, edge_attr, We1, We2, be2r, Wn1a)


_ROWS_PER_TILE = N // 16


@functools.lru_cache(maxsize=4)
def _build_k4(ne):
    nchunk = ne // CHUNK
    nj = (nchunk + NWORK - 1) // NWORK

    @functools.partial(
        pl.kernel,
        mesh=_sc_mesh(),
        out_type=jax.ShapeDtypeStruct((2 * N, HW), F32),
        scratch_types=[
            pltpu.VMEM((CHUNK,), jnp.int32),
            pltpu.VMEM((CHUNK, HW), F32),
            pltpu.VMEM_SHARED((N, HW), F32),
            pltpu.SemaphoreType.DMA,
        ],
        compiler_params=pltpu.CompilerParams(use_tc_tiling_on_sc=False),
    )
    def _k4_body(h3_hbm, col_hbm, zero_hbm, S2_hbm, coli, h3v, shared, s1):
        cid = lax.axis_index("c")
        sid = lax.axis_index("s")
        wid = sid * 2 + cid

        pltpu.sync_copy(zero_hbm.at[pl.ds(sid * _ROWS_PER_TILE, _ROWS_PER_TILE)],
                        shared.at[pl.ds(sid * _ROWS_PER_TILE, _ROWS_PER_TILE)])
        plsc.subcore_barrier()

        def body(j, carry):
            c = wid + NWORK * j

            @pl.when(c < nchunk)
            def _():
                base = c * CHUNK
                pltpu.sync_copy(col_hbm.at[pl.ds(base, CHUNK)], coli)
                pltpu.sync_copy(h3_hbm.at[pl.ds(base, CHUNK)], h3v)
                pltpu.sync_copy(h3v, shared.at[coli], add=True)

            return carry

        lax.fori_loop(0, nj, body, 0)
        plsc.subcore_barrier()
        pltpu.sync_copy(shared.at[pl.ds(sid * _ROWS_PER_TILE, _ROWS_PER_TILE)],
                        S2_hbm.at[pl.ds(cid * N + sid * _ROWS_PER_TILE,
                                        _ROWS_PER_TILE)])

    return _k4_body


def _k4_scatter(h3, col, zeros):
    return _build_k4(col.shape[0])(h3, col, zeros)


def _k5_body(x_ref, *rest):
    nS = 2 * NSLICE
    S_refs = rest[:nS]
    (batch_ref, u_ref, Wn1b_ref, bn1b_ref, Wn2a_ref, bn2a_ref, Wn2b_ref,
     bn2b_ref, Wg1_ref, bg1_ref, Wg2_ref, bg2_ref,
     xout_ref, uout_ref, acc_ref) = rest[nS:]
    i = pl.program_id(0)
    S = S_refs[0][...]
    for r in S_refs[1:]:
        S = S + r[...]
    cnt = S[:, DM:DM + 1]
    hmean = S[:, 0:DM] / jnp.maximum(cnt, 1.0)
    agg = jnp.dot(hmean, Wn1b_ref[...], preferred_element_type=F32) + bn1b_ref[...]
    agg = jnp.where(cnt > 0.0, agg, 0.0)

    bb = batch_ref[0, 0, :]
    oh = (bb[:, None] == lax.broadcasted_iota(jnp.int32, (BN, B), 1)).astype(F32)
    uw = jnp.dot(u_ref[...], Wn2a_ref[DN + DM:DN + DM + DG, :],
                 preferred_element_type=F32)
    h = jax.nn.relu(jnp.dot(x_ref[...], Wn2a_ref[0:DN, :],
                            preferred_element_type=F32)
                    + jnp.dot(agg, Wn2a_ref[DN:DN + DM, :],
                              preferred_element_type=F32)
                    + jnp.dot(oh, uw, preferred_element_type=F32)
                    + bn2a_ref[...])
    xo = jnp.dot(h, Wn2b_ref[...], preferred_element_type=F32) + bn2b_ref[...]
    xout_ref[...] = xo

    xext = jnp.concatenate([xo, jnp.ones((BN, 1), F32)], axis=1)
    ps = lax.dot_general(oh, xext, (((0,), (0,)), ((), ())),
                         preferred_element_type=F32)

    @pl.when(i == 0)
    def _():
        acc_ref[...] = jnp.zeros_like(acc_ref)

    acc_ref[...] += ps

    @pl.when(i == N // BN - 1)
    def _():
        acc = acc_ref[...]
        g = acc[:, 0:DM] / jnp.maximum(acc[:, DM:DM + 1], 1.0)
        hu = jax.nn.relu(jnp.dot(u_ref[...], Wg1_ref[0:DG, :],
                                 preferred_element_type=F32)
                         + jnp.dot(g, Wg1_ref[DG:DG + DM, :],
                                   preferred_element_type=F32)
                         + bg1_ref[...])
        uout_ref[...] = jnp.dot(hu, Wg2_ref[...],
                                preferred_element_type=F32) + bg2_ref[...]


def _k5(x, S_parts, batch3, u, Wn1b, bn1br, Wn2a, bn2ar, Wn2b, bn2br,
        Wg1, bg1r, Wg2, bg2r):
    full = lambda shape: pl.BlockSpec(shape, lambda i: tuple(0 for _ in shape))
    return pl.pallas_call(
        _k5_body,
        grid=(N // BN,),
        in_specs=[
            pl.BlockSpec((BN, DN), lambda i: (i, 0)),
            *[pl.BlockSpec((BN, HW), lambda i: (i, 0)) for _ in S_parts],
            pl.BlockSpec((1, 1, BN), lambda i: (i, 0, 0)),
            full((B, DG)),
            full((DM, DM)),
            full((1, DM)),
            full((DN + DM + DG, DM)),
            full((1, DM)),
            full((DM, DM)),
            full((1, DM)),
            full((DG + DM, DM)),
            full((1, DM)),
            full((DM, DM)),
            full((1, DM)),
        ],
        out_specs=[
            pl.BlockSpec((BN, DM), lambda i: (i, 0)),
            pl.BlockSpec((B, DM), lambda i: (0, 0)),
        ],
        out_shape=[
            jax.ShapeDtypeStruct((N, DM), F32),
            jax.ShapeDtypeStruct((B, DM), F32),
        ],
        scratch_shapes=[pltpu.VMEM((B, DM + 1), F32)],
    )(x, *S_parts, batch3, u, Wn1b, bn1br, Wn2a, bn2ar, Wn2b, bn2br,
      Wg1, bg1r, Wg2, bg2r)


def kernel(x, edge_index, edge_attr, u, batch,
           We1, be1, We2, be2,
           Wn1a, bn1a, Wn1b, bn1b,
           Wn2a, bn2a, Wn2b, bn2b,
           Wg1, bg1, Wg2, bg2):
    row = edge_index[0]
    col = edge_index[1]
    batch3 = batch.astype(jnp.int32).reshape(N // BN, 1, BN)
    r1 = lambda b: b.reshape(1, DM)

    T, btbl = _k1(x, batch3, u, We1, r1(be1), Wn1a, r1(bn1a))
    zeros = jnp.zeros((N, HW), F32)
    row32 = row.astype(jnp.int32)
    col32 = col.astype(jnp.int32)
    S_parts, e_outs = [], []
    for s in range(NSLICE):
        sl = slice(s * ES, (s + 1) * ES)
        G1 = _k2_gather(T, btbl, row32[sl], col32[sl])
        e_out_s, h3_s = _k3(G1, edge_attr[sl], We1, We2, r1(be2), Wn1a, s)
        S2_s = _k4_scatter(h3_s, col32[sl], zeros)
        e_outs.append(e_out_s)
        S_parts.extend([S2_s[:N], S2_s[N:]])
    e_out = jnp.concatenate(e_outs, axis=0)
    x_out, u_out = _k5(x, S_parts, batch3, u,
                       Wn1b, r1(bn1b), Wn2a, r1(bn2a), Wn2b, r1(bn2b),
                       Wg1, r1(bg1), Wg2, r1(bg2))
    return (x_out, e_out, u_out)


# speedup vs baseline: 1.0438x; 1.0168x over previous
import functools

import jax
import jax.numpy as jnp
from jax import lax
from jax.experimental import pallas as pl
from jax.experimental.pallas import tpu as pltpu
from jax.experimental.pallas import tpu_sc as plsc

F32 = jnp.float32
BF16 = jnp.bfloat16

N = 10000
E = 320000
B = 16
DN = 128
DE = 16
DG = 16
DM = 64
HW = 80

CHUNK = 128
NWORK = 32
NSLICE = 2
ES = E // NSLICE

BN = 2000
BE = 2000

@functools.lru_cache(maxsize=1)
def _sc_mesh():
    return plsc.VectorSubcoreMesh(core_axis_name="c", subcore_axis_name="s")


def _k1_body(x_ref, batch_ref, u_ref, We1_ref, be1_ref, Wn1a_ref, bn1a_ref,
             T_ref, btbl_ref):
    xb = x_ref[...]
    bb = batch_ref[0, 0, :]
    oh = (bb[:, None] == lax.broadcasted_iota(jnp.int32, (BN, B), 1)).astype(F32)
    uw = jnp.dot(u_ref[...], We1_ref[2 * DN + DE:2 * DN + DE + DG, :],
                 preferred_element_type=F32)
    Ta = (jnp.dot(xb, We1_ref[0:DN, :], preferred_element_type=F32)
          + jnp.dot(oh, uw, preferred_element_type=F32) + be1_ref[...])
    Tp = jnp.dot(xb, Wn1a_ref[0:DN, :], preferred_element_type=F32) + bn1a_ref[...]
    T_ref[...] = jnp.concatenate([Ta, Tp], axis=1)
    bt = jnp.dot(xb, We1_ref[DN:2 * DN, :], preferred_element_type=F32)
    btbl_ref[...] = jnp.concatenate([bt, jnp.zeros((BN, DM), F32)], axis=1)


def _k1(x, batch3, u, We1, be1r, Wn1a, bn1ar):
    full = lambda shape: pl.BlockSpec(shape, lambda i: tuple(0 for _ in shape))
    return pl.pallas_call(
        _k1_body,
        grid=(N // BN,),
        in_specs=[
            pl.BlockSpec((BN, DN), lambda i: (i, 0)),
            pl.BlockSpec((1, 1, BN), lambda i: (i, 0, 0)),
            full((B, DG)),
            full((2 * DN + DE + DG, DM)),
            full((1, DM)),
            full((DM + DN, DM)),
            full((1, DM)),
        ],
        out_specs=[
            pl.BlockSpec((BN, 2 * DM), lambda i: (i, 0)),
            pl.BlockSpec((BN, 2 * DM), lambda i: (i, 0)),
        ],
        out_shape=[
            jax.ShapeDtypeStruct((N, 2 * DM), F32),
            jax.ShapeDtypeStruct((N, 2 * DM), F32),
        ],
    )(x, batch3, u, We1, be1r, Wn1a, bn1ar)


@functools.lru_cache(maxsize=4)
def _build_k2(ne):
    nchunk = ne // CHUNK
    nj = (nchunk + NWORK - 1) // NWORK

    @functools.partial(
        pl.kernel,
        mesh=_sc_mesh(),
        out_type=jax.ShapeDtypeStruct((ne, 2 * DM), F32),
        scratch_types=[
            pltpu.VMEM((2, CHUNK), jnp.int32),
            pltpu.VMEM((2, CHUNK), jnp.int32),
            pltpu.VMEM((2, CHUNK, 2 * DM), F32),
            pltpu.VMEM((2, CHUNK, 2 * DM), F32),
            pltpu.SemaphoreType.DMA,
            pltpu.SemaphoreType.DMA,
            pltpu.SemaphoreType.DMA,
            pltpu.SemaphoreType.DMA,
        ],
    )
    def _k2_body(T_hbm, btbl_hbm, row_hbm, col_hbm, G1_hbm,
                 rowi, coli, Tg, bg, sT0, sT1, sb0, sb1):
        wid = lax.axis_index("s") * 2 + lax.axis_index("c")
        sT = (sT0, sT1)
        sb = (sb0, sb1)

        def start(j, b):
            c = wid + NWORK * j

            @pl.when(c < nchunk)
            def _():
                base = c * CHUNK
                pltpu.sync_copy(row_hbm.at[pl.ds(base, CHUNK)], rowi.at[b])
                pltpu.sync_copy(col_hbm.at[pl.ds(base, CHUNK)], coli.at[b])
                pltpu.async_copy(T_hbm.at[rowi.at[b]], Tg.at[b], sT[b])
                pltpu.async_copy(btbl_hbm.at[coli.at[b]], bg.at[b], sb[b])

        def finish(j, b):
            c = wid + NWORK * j

            @pl.when(c < nchunk)
            def _():
                base = c * CHUNK
                pltpu.make_async_copy(T_hbm.at[rowi.at[b]], Tg.at[b],
                                      sT[b]).wait()
                pltpu.make_async_copy(btbl_hbm.at[coli.at[b]], bg.at[b],
                                      sb[b]).wait()

                def add_row(i, carry2):
                    for q in range(DM // 16):
                        sl = pl.ds(q * 16, 16)
                        Tg[b, i, sl] += bg[b, i, sl]
                    return carry2

                lax.fori_loop(0, CHUNK, add_row, 0)
                pltpu.sync_copy(Tg.at[b], G1_hbm.at[pl.ds(base, CHUNK)])

        start(0, 0)

        def body(jj, carry):
            j0 = 2 * jj
            start(j0 + 1, 1)
            finish(j0, 0)
            start(j0 + 2, 0)
            finish(j0 + 1, 1)
            return carry

        lax.fori_loop(0, (nj + 1) // 2, body, 0)

    return _k2_body


def _k2_gather(T, btbl, row, col):
    return _build_k2(row.shape[0])(T, btbl, row, col)


def _k3_body(G1_ref, ea_ref, We1_ref, We2_ref, be2_ref, Wn1a_ref,
             eout_ref, h3_ref):
    g1 = G1_ref[...]
    h1 = jax.nn.relu(g1[:, 0:DM]
                     + jnp.dot(ea_ref[...], We1_ref[2 * DN:2 * DN + DE, :],
                               preferred_element_type=F32))
    eo = jnp.dot(h1, We2_ref[...], preferred_element_type=F32) + be2_ref[...]
    eout_ref[...] = eo
    h2 = jax.nn.relu(g1[:, DM:2 * DM]
                     + jnp.dot(eo, Wn1a_ref[DN:DN + DM, :],
                               preferred_element_type=F32))
    pat = (lax.broadcasted_iota(jnp.int32, (BE, HW - DM), 1) == 0).astype(F32)
    h3_ref[...] = jnp.concatenate([h2, jnp.broadcast_to(pat, (BE, HW - DM))],
                                  axis=1)


def _k3(G1, edge_attr, We1, We2, be2r, Wn1a, s):
    ne = G1.shape[0]
    nb = ne // BE
    full = lambda shape: pl.BlockSpec(shape, lambda i: tuple(0 for _ in shape))
    return pl.pallas_call(
        _k3_body,
        grid=(nb,),
        in_specs=[
            pl.BlockSpec((BE, 2 * DM), lambda i: (i, 0)),
            pl.BlockSpec((BE, DE), lambda i: (i, 0)),
            full((2 * DN + DE + DG, DM)),
            full((DM, DM)),
            full((1, DM)),
            full((DM + DN, DM)),
        ],
        out_specs=[
            pl.BlockSpec((BE, DM), lambda i: (i, 0)),
            pl.BlockSpec((BE, HW), lambda i: (i, 0)),
        ],
        out_shape=[
            jax.ShapeDtypeStruct((ne, DM), F32),
            jax.ShapeDtypeStruct((ne, HW), F32),
        ],
    )(G1, edge_attr, We1, We2, be2r, Wn1a)


_ROWS_PER_TILE = N // 16


@functools.lru_cache(maxsize=4)
def _build_k4(ne):
    nchunk = ne // CHUNK
    nj = (nchunk + NWORK - 1) // NWORK

    @functools.partial(
        pl.kernel,
        mesh=_sc_mesh(),
        out_type=jax.ShapeDtypeStruct((2 * N, HW), F32),
        scratch_types=[
            pltpu.VMEM((CHUNK,), jnp.int32),
            pltpu.VMEM((CHUNK, HW), F32),
            pltpu.VMEM_SHARED((N, HW), F32),
            pltpu.SemaphoreType.DMA,
        ],
        compiler_params=pltpu.CompilerParams(use_tc_tiling_on_sc=False),
    )
    def _k4_body(h3_hbm, col_hbm, zero_hbm, S2_hbm, coli, h3v, shared, s1):
        cid = lax.axis_index("c")
        sid = lax.axis_index("s")
        wid = sid * 2 + cid

        pltpu.sync_copy(zero_hbm.at[pl.ds(sid * _ROWS_PER_TILE, _ROWS_PER_TILE)],
                        shared.at[pl.ds(sid * _ROWS_PER_TILE, _ROWS_PER_TILE)])
        plsc.subcore_barrier()

        def body(j, carry):
            c = wid + NWORK * j

            @pl.when(c < nchunk)
            def _():
                base = c * CHUNK
                pltpu.sync_copy(col_hbm.at[pl.ds(base, CHUNK)], coli)
                pltpu.sync_copy(h3_hbm.at[pl.ds(base, CHUNK)], h3v)
                pltpu.sync_copy(h3v, shared.at[coli], add=True)

            return carry

        lax.fori_loop(0, nj, body, 0)
        plsc.subcore_barrier()
        pltpu.sync_copy(shared.at[pl.ds(sid * _ROWS_PER_TILE, _ROWS_PER_TILE)],
                        S2_hbm.at[pl.ds(cid * N + sid * _ROWS_PER_TILE,
                                        _ROWS_PER_TILE)])

    return _k4_body


def _k4_scatter(h3, col, zeros):
    return _build_k4(col.shape[0])(h3, col, zeros)


def _k5_body(x_ref, *rest):
    nS = 2 * NSLICE
    S_refs = rest[:nS]
    (batch_ref, u_ref, Wn1b_ref, bn1b_ref, Wn2a_ref, bn2a_ref, Wn2b_ref,
     bn2b_ref, Wg1_ref, bg1_ref, Wg2_ref, bg2_ref,
     xout_ref, uout_ref, acc_ref) = rest[nS:]
    i = pl.program_id(0)
    S = S_refs[0][...]
    for r in S_refs[1:]:
        S = S + r[...]
    cnt = S[:, DM:DM + 1]
    hmean = S[:, 0:DM] / jnp.maximum(cnt, 1.0)
    agg = jnp.dot(hmean, Wn1b_ref[...], preferred_element_type=F32) + bn1b_ref[...]
    agg = jnp.where(cnt > 0.0, agg, 0.0)

    bb = batch_ref[0, 0, :]
    oh = (bb[:, None] == lax.broadcasted_iota(jnp.int32, (BN, B), 1)).astype(F32)
    uw = jnp.dot(u_ref[...], Wn2a_ref[DN + DM:DN + DM + DG, :],
                 preferred_element_type=F32)
    h = jax.nn.relu(jnp.dot(x_ref[...], Wn2a_ref[0:DN, :],
                            preferred_element_type=F32)
                    + jnp.dot(agg, Wn2a_ref[DN:DN + DM, :],
                              preferred_element_type=F32)
                    + jnp.dot(oh, uw, preferred_element_type=F32)
                    + bn2a_ref[...])
    xo = jnp.dot(h, Wn2b_ref[...], preferred_element_type=F32) + bn2b_ref[...]
    xout_ref[...] = xo

    xext = jnp.concatenate([xo, jnp.ones((BN, 1), F32)], axis=1)
    ps = lax.dot_general(oh, xext, (((0,), (0,)), ((), ())),
                         preferred_element_type=F32)

    @pl.when(i == 0)
    def _():
        acc_ref[...] = jnp.zeros_like(acc_ref)

    acc_ref[...] += ps

    @pl.when(i == N // BN - 1)
    def _():
        acc = acc_ref[...]
        g = acc[:, 0:DM] / jnp.maximum(acc[:, DM:DM + 1], 1.0)
        hu = jax.nn.relu(jnp.dot(u_ref[...], Wg1_ref[0:DG, :],
                                 preferred_element_type=F32)
                         + jnp.dot(g, Wg1_ref[DG:DG + DM, :],
                                   preferred_element_type=F32)
                         + bg1_ref[...])
        uout_ref[...] = jnp.dot(hu, Wg2_ref[...],
                                preferred_element_type=F32) + bg2_ref[...]


def _k5(x, S_parts, batch3, u, Wn1b, bn1br, Wn2a, bn2ar, Wn2b, bn2br,
        Wg1, bg1r, Wg2, bg2r):
    full = lambda shape: pl.BlockSpec(shape, lambda i: tuple(0 for _ in shape))
    return pl.pallas_call(
        _k5_body,
        grid=(N // BN,),
        in_specs=[
            pl.BlockSpec((BN, DN), lambda i: (i, 0)),
            *[pl.BlockSpec((BN, HW), lambda i: (i, 0)) for _ in S_parts],
            pl.BlockSpec((1, 1, BN), lambda i: (i, 0, 0)),
            full((B, DG)),
            full((DM, DM)),
            full((1, DM)),
            full((DN + DM + DG, DM)),
            full((1, DM)),
            full((DM, DM)),
            full((1, DM)),
            full((DG + DM, DM)),
            full((1, DM)),
            full((DM, DM)),
            full((1, DM)),
        ],
        out_specs=[
            pl.BlockSpec((BN, DM), lambda i: (i, 0)),
            pl.BlockSpec((B, DM), lambda i: (0, 0)),
        ],
        out_shape=[
            jax.ShapeDtypeStruct((N, DM), F32),
            jax.ShapeDtypeStruct((B, DM), F32),
        ],
        scratch_shapes=[pltpu.VMEM((B, DM + 1), F32)],
    )(x, *S_parts, batch3, u, Wn1b, bn1br, Wn2a, bn2ar, Wn2b, bn2br,
      Wg1, bg1r, Wg2, bg2r)


def kernel(x, edge_index, edge_attr, u, batch,
           We1, be1, We2, be2,
           Wn1a, bn1a, Wn1b, bn1b,
           Wn2a, bn2a, Wn2b, bn2b,
           Wg1, bg1, Wg2, bg2):
    row = edge_index[0]
    col = edge_index[1]
    batch3 = batch.astype(jnp.int32).reshape(N // BN, 1, BN)
    r1 = lambda b: b.reshape(1, DM)

    T, btbl = _k1(x, batch3, u, We1, r1(be1), Wn1a, r1(bn1a))
    zeros = jnp.zeros((N, HW), F32)
    row32 = row.astype(jnp.int32)
    col32 = col.astype(jnp.int32)
    S_parts, e_outs = [], []
    for s in range(NSLICE):
        sl = slice(s * ES, (s + 1) * ES)
        G1 = _k2_gather(T, btbl, row32[sl], col32[sl])
        e_out_s, h3_s = _k3(G1, edge_attr[sl], We1, We2, r1(be2), Wn1a, s)
        S2_s = _k4_scatter(h3_s, col32[sl], zeros)
        e_outs.append(e_out_s)
        S_parts.extend([S2_s[:N], S2_s[N:]])
    e_out = jnp.concatenate(e_outs, axis=0)
    x_out, u_out = _k5(x, S_parts, batch3, u,
                       Wn1b, r1(bn1b), Wn2a, r1(bn2a), Wn2b, r1(bn2b),
                       Wg1, r1(bg1), Wg2, r1(bg2))
    return (x_out, e_out, u_out)
